# baseline (speedup 1.0000x reference)
import functools

import jax
import jax.numpy as jnp
from jax import lax
from jax.experimental import pallas as pl
from jax.experimental.pallas import tpu as pltpu
from jax.experimental.pallas import tpu_sc as plsc

BATCH = 4096
NSTEP = 11
LENGTH = 20
INPUT_SIZE = 64
ROW = LENGTH * INPUT_SIZE

NC = 2
NS = 16
NW = NC * NS
B_PER_W = BATCH // NW
CHUNK = 64
NCHUNK = B_PER_W // CHUNK
LANES = 16


def _gather_rows(table, steps3, base3):
    mesh = plsc.VectorSubcoreMesh(core_axis_name="c", subcore_axis_name="s")

    @functools.partial(
        pl.kernel,
        mesh=mesh,
        out_type=jax.ShapeDtypeStruct((BATCH, ROW), jnp.float32),
        scratch_types=[
            pltpu.VMEM((NCHUNK, CHUNK), jnp.int32),
            pltpu.VMEM((NCHUNK, CHUNK), jnp.int32),
            pltpu.VMEM((NCHUNK, CHUNK), jnp.int32),
            pltpu.VMEM((CHUNK, ROW), jnp.float32),
            pltpu.SemaphoreType.DMA,
        ],
    )
    def k(table_hbm, steps_hbm, base_hbm, out_hbm, steps_v, base_v, idx_v,
          rows_v, sem):
        wid = lax.axis_index("s") * NC + lax.axis_index("c")
        start = wid * B_PER_W
        pltpu.sync_copy(steps_hbm.at[wid], steps_v)
        pltpu.sync_copy(base_hbm.at[wid], base_v)

        for c in range(NCHUNK):
            @pl.loop(0, CHUNK, step=LANES)
            def _(i, c=c):
                sl = (c, pl.ds(i, LANES))
                idx_v[sl] = steps_v[sl] + base_v[sl]

        for c in range(NCHUNK):
            pltpu.async_copy(table_hbm.at[idx_v.at[c]], rows_v, sem).wait()
            pltpu.sync_copy(rows_v, out_hbm.at[pl.ds(start + c * CHUNK, CHUNK)])

    return k(table, steps3, base3)


def kernel(embeddings, conditions, steps, weight, lengths):
    table = weight.reshape(BATCH * NSTEP, ROW)
    steps3 = steps.astype(jnp.int32).reshape(NW, NCHUNK, CHUNK)
    base3 = (jnp.arange(BATCH, dtype=jnp.int32) * NSTEP).reshape(
        NW, NCHUNK, CHUNK)
    out = _gather_rows(table, steps3, base3)
    return (out.reshape(BATCH, LENGTH, INPUT_SIZE), lengths)


# speedup vs baseline: 2.3475x; 2.3475x over previous
import functools

import jax
import jax.numpy as jnp
from jax import lax
from jax.experimental import pallas as pl
from jax.experimental.pallas import tpu as pltpu
from jax.experimental.pallas import tpu_sc as plsc

BATCH = 4096
NSTEP = 11
LENGTH = 20
INPUT_SIZE = 64

NC = 2
NS = 16
NW = NC * NS
B_PER_W = BATCH // NW
GROUP = 16
NGROUP = B_PER_W // GROUP


def _gather_rows(table, steps):
    mesh = plsc.VectorSubcoreMesh(core_axis_name="c", subcore_axis_name="s")

    @functools.partial(
        pl.kernel,
        mesh=mesh,
        out_type=jax.ShapeDtypeStruct((BATCH, LENGTH, INPUT_SIZE),
                                      jnp.float32),
        scratch_types=[
            pltpu.VMEM((B_PER_W,), jnp.int32),
            pltpu.VMEM((GROUP, LENGTH, INPUT_SIZE), jnp.float32),
            pltpu.SemaphoreType.DMA,
        ],
    )
    def k(table_hbm, steps_hbm, out_hbm, steps_v, rows_v, sem):
        wid = lax.axis_index("s") * NC + lax.axis_index("c")
        start = wid * B_PER_W
        pltpu.sync_copy(steps_hbm.at[pl.ds(start, B_PER_W)], steps_v)

        @pl.loop(0, NGROUP)
        def _(g):
            base = g * GROUP
            svec = steps_v[pl.ds(base, GROUP)]
            copies = []
            for j in range(GROUP):
                idx = (start + base + j) * NSTEP + svec[j]
                copies.append(
                    pltpu.make_async_copy(table_hbm.at[idx], rows_v.at[j],
                                          sem))
            for c in copies:
                c.start()
            for c in copies:
                c.wait()
            pltpu.sync_copy(rows_v,
                            out_hbm.at[pl.ds(start + base, GROUP)])

    return k(table, steps)


def kernel(embeddings, conditions, steps, weight, lengths):
    table = weight.reshape(BATCH * NSTEP, LENGTH, INPUT_SIZE)
    steps32 = steps.astype(jnp.int32)
    out = _gather_rows(table, steps32)
    return (out, lengths)
